# baseline (speedup 1.0000x reference)
import functools

import jax
import jax.numpy as jnp
from jax import lax
from jax.experimental import pallas as pl
from jax.experimental.pallas import tpu as pltpu
from jax.experimental.pallas import tpu_sc as plsc

TOKEN_BLOCK = 1024
TOKENS = 8192
EXPERTS = 64
K = TOKENS // EXPERTS
NWORKERS = 32
LANES = 16


def _mm_body(x_ref, w_ref, logits_ref, pT_ref, ent_ref, ppack_ref):
    j = pl.program_id(0)
    nblk = pl.num_programs(0)
    x = x_ref[...]
    w = w_ref[...]
    logits = jax.lax.dot_general(
        x, w, (((1,), (1,)), ((), ())), preferred_element_type=jnp.float32
    )
    logits_ref[...] = logits
    m = jnp.max(logits, axis=-1, keepdims=True)
    unnorm = jnp.exp(logits - m)
    p = unnorm / jnp.sum(unnorm, axis=-1, keepdims=True)
    pT_ref[...] = jnp.transpose(p, (1, 0))
    blk = logits.shape[0]
    half_blocks = nblk // 2
    row = jnp.where(j < half_blocks, j, j - half_blocks) * blk

    @pl.when(j < half_blocks)
    def _():
        ppack_ref[pl.ds(row, blk), 0:64] = p

    @pl.when(j >= half_blocks)
    def _():
        ppack_ref[pl.ds(row, blk), 64:128] = p

    ent = -jnp.sum(p * jnp.log(p + 1e-8))

    @pl.when(j == 0)
    def _():
        ent_ref[...] = jnp.zeros_like(ent_ref)

    ent_ref[...] += jnp.reshape(ent, (1, 1))


def _splat(x):
    return jnp.full((LANES,), x, jnp.int32)


def _ffs(hit):
    c = plsc.cumsum(hit.astype(jnp.int32))
    return jnp.sum((c == 0).astype(jnp.int32), axis=0)


def _lane_get(v, i):
    lanes = lax.iota(jnp.int32, LANES)
    return jnp.sum(jnp.where(lanes == i, v, 0), axis=0)


def _sc_routing_kernel(pT_hbm, thr_hbm, col_v, hist_v, tmp_v):
    n_iter = TOKENS // LANES
    lanes = lax.iota(jnp.int32, LANES)
    ones = jnp.ones((LANES,), jnp.int32)
    wid = lax.axis_index("s") * 2 + lax.axis_index("c")

    def zero_hist(nwords):
        def zb(i, _):
            hist_v[pl.ds(i * LANES, LANES)] = jnp.zeros((LANES,), jnp.int32)
            return 0
        lax.fori_loop(0, nwords // LANES, zb, 0)

    def scan_hist(nbins, need, descending):
        nch = nbins // LANES

        def body(c, carry):
            bstar, prev, running, found = carry
            if descending:
                start = nbins - LANES * (c + 1)
            else:
                start = c * LANES

            def fold_bin(bb, a):
                s = jnp.sum(hist_v[pl.ds((start + bb) * LANES, LANES)],
                            axis=0)
                return a + jnp.where(lanes == bb, s, 0)

            acc = lax.fori_loop(0, LANES, fold_bin,
                                jnp.zeros((LANES,), jnp.int32))
            h = acc[::-1] if descending else acc
            cs = plsc.cumsum(h) + running
            hit = cs >= need
            f = _ffs(hit)
            has = f < LANES
            fc = jnp.minimum(f, LANES - 1)
            if descending:
                binv = start + (LANES - 1) - fc
            else:
                binv = start + fc
            csf = _lane_get(cs, fc)
            hf = _lane_get(h, fc)
            newly = has & jnp.logical_not(found)
            bstar = jnp.where(newly, binv, bstar)
            prev = jnp.where(newly, csf - hf, prev)
            running = _lane_get(cs, LANES - 1)
            found = found | has
            return bstar, prev, running, found

        z = jnp.int32(0)
        bstar, prev, _, _ = lax.fori_loop(
            0, nch, body, (z, z, z, jnp.bool_(False)))
        return bstar, prev

    def hist_pass(shift, nbins, sel_shift, sel_val, use_mask):
        def body(i, _):
            v = col_v[pl.ds(i * LANES, LANES)]
            bits = lax.bitcast_convert_type(v, jnp.int32)
            b = (bits >> shift) & jnp.int32(nbins - 1)
            idx = b * LANES + lanes
            if use_mask:
                ok = (bits >> sel_shift) == sel_val
            else:
                ok = bits >= jnp.int32(0)
            plsc.addupdate_scatter(hist_v, [idx], ones, mask=ok)
            return 0
        lax.fori_loop(0, n_iter, body, 0)

    def idx_hist_pass(shift, nbins, tbits, sel_shift, sel_val, use_sel):
        def body(i, _):
            v = col_v[pl.ds(i * LANES, LANES)]
            bits = lax.bitcast_convert_type(v, jnp.int32)
            tok = i * LANES + lanes
            b = (tok >> shift) & jnp.int32(nbins - 1)
            idx = b * LANES + lanes
            ok = bits == tbits
            if use_sel:
                ok = ok & ((tok >> sel_shift) == sel_val)
            plsc.addupdate_scatter(hist_v, [idx], ones, mask=ok)
            return 0
        lax.fori_loop(0, n_iter, body, 0)

    for sub in range(2):
        eidx = wid * 2 + sub
        pltpu.sync_copy(pT_hbm.at[pl.ds(eidx * TOKENS, TOKENS)], col_v)

        kneed = jnp.int32(K)
        prefix = jnp.int32(0)
        m_above = jnp.int32(0)
        for (shift, width) in ((22, 8), (14, 8), (6, 8), (0, 6)):
            nbins = 1 << width
            zero_hist(nbins * LANES)
            use_mask = shift != 22
            hist_pass(shift, nbins, shift + width, prefix, use_mask)
            need = kneed - m_above
            bstar, prev = scan_hist(nbins, need, True)
            prefix = prefix * nbins + bstar
            m_above = m_above + prev
        tbits = prefix
        r = kneed - m_above

        zero_hist(256 * LANES)
        idx_hist_pass(5, 256, tbits, 0, jnp.int32(0), False)
        bA, prevA = scan_hist(256, r, False)
        zero_hist(32 * LANES)
        idx_hist_pass(0, 32, tbits, 5, bA, True)
        bB, _ = scan_hist(32, r - prevA, False)
        istar = bA * 32 + bB

        out = jnp.where(lanes == 0, tbits,
                        jnp.where(lanes == 1, istar, 0))
        tmp_v[...] = out
        pltpu.sync_copy(tmp_v, thr_hbm.at[pl.ds(eidx * LANES, LANES)])


def _sc_routing(pT):
    kfn = pl.kernel(
        _sc_routing_kernel,
        out_type=jax.ShapeDtypeStruct((EXPERTS * LANES,), jnp.int32),
        mesh=plsc.VectorSubcoreMesh(core_axis_name="c", subcore_axis_name="s"),
        scratch_types=[
            pltpu.VMEM((TOKENS,), jnp.float32),
            pltpu.VMEM((256 * LANES,), jnp.int32),
            pltpu.VMEM((LANES,), jnp.int32),
        ],
        compiler_params=pltpu.CompilerParams(needs_layout_passes=False),
    )
    return kfn(pT.reshape(-1)).reshape(EXPERTS, LANES)


def _merge_body(ppack_ref, thr_ref, sel_ref, w_out_ref, var_ref):
    half, twoe = ppack_ref.shape
    tokens = half * 2
    experts = twoe // 2
    pw = ppack_ref[...]
    pb = lax.bitcast_convert_type(pw, jnp.int32)
    tb = jnp.transpose(thr_ref[:, 0:1], (1, 0))
    istar = jnp.transpose(thr_ref[:, 1:2], (1, 0))

    def dup(v):
        return jnp.concatenate([v, v], axis=1)

    t128 = dup(tb)
    gt = pb > t128
    tie = pb == t128

    laneio = lax.broadcasted_iota(jnp.int32, (half, twoe), 1)
    hi_half = laneio >= experts
    idxw = (lax.broadcasted_iota(jnp.int32, (half, twoe), 0)
            + jnp.where(hi_half, half, 0))

    sel_mask = gt | (tie & (idxw <= dup(istar)))

    eio = jnp.where(hi_half, laneio - experts, laneio)
    neg = jnp.int32(-1)
    e_lo = jnp.max(jnp.where(sel_mask & ~hi_half, eio, neg), axis=1,
                   keepdims=True)
    e_hi = jnp.max(jnp.where(sel_mask & hi_half, eio, neg), axis=1,
                   keepdims=True)
    w_lo = jnp.max(jnp.where(sel_mask & ~hi_half & (eio == e_lo), pw, 0.0),
                   axis=1, keepdims=True)
    w_hi = jnp.max(jnp.where(sel_mask & hi_half & (eio == e_hi), pw, 0.0),
                   axis=1, keepdims=True)
    sel_lo = jnp.maximum(e_lo, 0)
    sel_hi = jnp.maximum(e_hi, 0)

    selw = jnp.where(hi_half, sel_hi, sel_lo)
    counts = (jnp.sum((selw == eio).astype(jnp.float32), axis=0,
                      keepdims=True))
    counts = counts[:, :experts] + counts[:, experts:]
    load = counts / jnp.float32(tokens)
    mu = jnp.sum(load) / jnp.float32(experts)
    var = jnp.sum((load - mu) ** 2) / jnp.float32(experts - 1)

    sel_ref[0:half, :] = sel_lo
    sel_ref[half:tokens, :] = sel_hi
    w_out_ref[0:half, :] = w_lo
    w_out_ref[half:tokens, :] = w_hi
    var_ref[...] = jnp.reshape(var, (1, 1))


@jax.jit
def kernel(hidden_states, W):
    tokens, hidden = hidden_states.shape
    experts = W.shape[0]
    nblk = tokens // TOKEN_BLOCK

    logits, pT, ent_sum, ppack = pl.pallas_call(
        _mm_body,
        grid=(nblk,),
        in_specs=[
            pl.BlockSpec((TOKEN_BLOCK, hidden), lambda j: (j, 0)),
            pl.BlockSpec((experts, hidden), lambda j: (0, 0)),
        ],
        out_specs=[
            pl.BlockSpec((TOKEN_BLOCK, experts), lambda j: (j, 0)),
            pl.BlockSpec((experts, TOKEN_BLOCK), lambda j: (0, j)),
            pl.BlockSpec((1, 1), lambda j: (0, 0)),
            pl.BlockSpec((tokens // 2, 2 * experts), lambda j: (0, 0)),
        ],
        out_shape=[
            jax.ShapeDtypeStruct((tokens, experts), jnp.float32),
            jax.ShapeDtypeStruct((experts, tokens), jnp.float32),
            jax.ShapeDtypeStruct((1, 1), jnp.float32),
            jax.ShapeDtypeStruct((tokens // 2, 2 * experts), jnp.float32),
        ],
    )(hidden_states, W)

    thr = _sc_routing(pT)

    sel, wts, var = pl.pallas_call(
        _merge_body,
        out_shape=[
            jax.ShapeDtypeStruct((tokens, 1), jnp.int32),
            jax.ShapeDtypeStruct((tokens, 1), jnp.float32),
            jax.ShapeDtypeStruct((1, 1), jnp.float32),
        ],
    )(ppack, thr)

    entropy = ent_sum[0, 0] / jnp.float32(tokens)
    return (logits, sel, wts, var[0, 0], entropy)
